# Initial kernel scaffold; baseline (speedup 1.0000x reference)
import functools

import jax
import jax.numpy as jnp
from jax import lax
from jax.experimental import pallas as pl
from jax.experimental.pallas import tpu as pltpu
from jax.experimental.pallas import tpu_sc as plsc

N = 10000
E = 320000
D = 128
R = 8
NC = 2
NS = 16
NW = NC * NS
L = 16
NPAD = 10240
SEG = NPAD // NW
CH = E // NW
W = 80
NWIN = CH // W
NEG = -3.0e38

_mesh = plsc.VectorSubcoreMesh(core_axis_name="c", subcore_axis_name="s")


def _wid():
    return lax.axis_index("s") * NC + lax.axis_index("c")


def _proj_body(h_ref, w_ref, o_ref):
    o_ref[...] = jnp.dot(h_ref[...], w_ref[0],
                         preferred_element_type=jnp.float32)


def _all_proj(h, rw):
    BR = 1000
    G = N // BR
    return pl.pallas_call(
        _proj_body,
        grid=(R, G),
        in_specs=[
            pl.BlockSpec((BR, D), lambda r, i: (i, 0)),
            pl.BlockSpec((1, D, D), lambda r, i: (r, 0, 0)),
        ],
        out_specs=pl.BlockSpec((BR, D), lambda r, i: (r * G + i, 0)),
        out_shape=jax.ShapeDtypeStruct((R * N, D), jnp.float32),
    )(h, rw)


@functools.partial(
    pl.kernel,
    out_type=jax.ShapeDtypeStruct((E,), jnp.float32),
    mesh=_mesh,
    scratch_types=[
        pltpu.VMEM((W,), jnp.int32),
        pltpu.VMEM((W,), jnp.int32),
        pltpu.VMEM((W,), jnp.int32),
        pltpu.VMEM((W,), jnp.int32),
        pltpu.VMEM((W,), jnp.int32),
        pltpu.VMEM((W, D), jnp.float32),
        pltpu.VMEM((W, D), jnp.float32),
        pltpu.VMEM((W, D), jnp.float32),
        pltpu.VMEM((W,), jnp.float32),
        pltpu.SemaphoreType.DMA,
    ],
)
def _att_kernel(ap_hbm, ef_hbm, src_hbm, dst_hbm, et_hbm, att_hbm,
                src_v, dst_v, et_v, fsrc_v, fdst_v, t_v, hr_v, ef_v,
                att_v, sem):
    wid = _wid()
    base0 = wid * CH
    iota = lax.iota(jnp.int32, L)

    @pl.loop(0, NWIN)
    def _win(w):
        base = base0 + w * W
        pltpu.sync_copy(src_hbm.at[pl.ds(base, W)], src_v)
        pltpu.sync_copy(dst_hbm.at[pl.ds(base, W)], dst_v)
        pltpu.sync_copy(et_hbm.at[pl.ds(base, W)], et_v)

        @pl.loop(0, W // L)
        def _idx(k):
            sl = pl.ds(k * L, L)
            et = et_v[sl]
            fsrc_v[sl] = et * N + src_v[sl]
            fdst_v[sl] = et * N + dst_v[sl]

        cp1 = pltpu.async_copy(ap_hbm.at[fsrc_v], t_v, sem)
        cp2 = pltpu.async_copy(ap_hbm.at[fdst_v], hr_v, sem)
        cp3 = pltpu.async_copy(ef_hbm.at[pl.ds(base, W)], ef_v, sem)
        cp1.wait()
        cp2.wait()
        cp3.wait()

        @pl.loop(0, W // L)
        def _blk(k):
            acc = jnp.zeros((L,), jnp.float32)
            for j in range(L):
                e = k * L + j
                p = jnp.zeros((L,), jnp.float32)
                for q in range(D // L):
                    sl = pl.ds(q * L, L)
                    t = t_v[e, sl]
                    x = hr_v[e, sl] + ef_v[e, sl]
                    th = 1.0 - 2.0 / (jnp.exp(2.0 * x) + 1.0)
                    p = p + t * th
                s = jnp.sum(p)
                acc = jnp.where(iota == j, s, acc)
            att_v[pl.ds(k * L, L)] = acc

        pltpu.sync_copy(att_v, att_hbm.at[pl.ds(base, W)])


@functools.partial(
    pl.kernel,
    out_type=jax.ShapeDtypeStruct((NW, NPAD), jnp.float32),
    mesh=_mesh,
    scratch_types=[
        pltpu.VMEM((NPAD,), jnp.float32),
        pltpu.VMEM((W,), jnp.float32),
        pltpu.VMEM((W,), jnp.int32),
        pltpu.SemaphoreType.DMA,
    ],
)
def _amax_parts_kernel(att_hbm, dst_hbm, out_hbm, tab_v, att_v, dst_v, sem):
    wid = _wid()
    base0 = wid * CH

    @pl.loop(0, NPAD // L)
    def _init(i):
        tab_v[pl.ds(i * L, L)] = jnp.full((L,), NEG, jnp.float32)

    @pl.loop(0, NWIN)
    def _win(w):
        base = base0 + w * W
        pltpu.sync_copy(att_hbm.at[pl.ds(base, W)], att_v)
        pltpu.sync_copy(dst_hbm.at[pl.ds(base, W)], dst_v)

        @pl.loop(0, W // L)
        def _blk(k):
            sl = pl.ds(k * L, L)
            d = dst_v[sl]
            a = att_v[sl]

            def body(_):
                cur = plsc.load_gather(tab_v, [d])
                need = a > cur
                plsc.store_scatter(tab_v, [d], a, mask=need)
                return jnp.any(need)

            lax.while_loop(lambda go: go, body, jnp.bool_(True))

    pltpu.sync_copy(tab_v, out_hbm.at[wid])


@functools.partial(
    pl.kernel,
    out_type=jax.ShapeDtypeStruct((NPAD,), jnp.float32),
    mesh=_mesh,
    scratch_types=[
        pltpu.VMEM((NW, SEG), jnp.float32),
        pltpu.VMEM((SEG,), jnp.float32),
        pltpu.SemaphoreType.DMA,
    ],
)
def _amax_combine_kernel(parts_hbm, out_hbm, buf_v, res_v, sem):
    wid = _wid()
    col = wid * SEG
    for k in range(NW):
        pltpu.sync_copy(parts_hbm.at[k, pl.ds(col, SEG)], buf_v.at[k])

    @pl.loop(0, SEG // L)
    def _blk(t):
        sl = pl.ds(t * L, L)
        m = buf_v[0, sl]
        for k in range(1, NW):
            m = jnp.maximum(m, buf_v[k, sl])
        res_v[sl] = jnp.where(m > -1.0e38, m, 0.0)

    pltpu.sync_copy(res_v, out_hbm.at[pl.ds(col, SEG)])


ZB = 64

@functools.partial(
    pl.kernel,
    out_type=(
        jax.ShapeDtypeStruct((NC, NPAD, D), jnp.float32),
        jax.ShapeDtypeStruct((NC, NPAD, L), jnp.float32),
    ),
    mesh=_mesh,
    scratch_types=[
        pltpu.VMEM((NPAD,), jnp.float32),
        pltpu.VMEM((W,), jnp.int32),
        pltpu.VMEM((W,), jnp.int32),
        pltpu.VMEM((W,), jnp.float32),
        pltpu.VMEM((W,), jnp.float32),
        pltpu.VMEM((W, D), jnp.float32),
        pltpu.VMEM((W, L), jnp.float32),
        pltpu.VMEM((W // L, L), jnp.int32),
        pltpu.VMEM((ZB, D), jnp.float32),
        pltpu.VMEM((ZB, L), jnp.float32),
        pltpu.VMEM_SHARED((NPAD, D), jnp.float32),
        pltpu.VMEM_SHARED((NPAD, L), jnp.float32),
        pltpu.SemaphoreType.DMA,
    ],
)
def _aggregate_kernel(att_hbm, dst_hbm, src_hbm, amax_hbm, h_hbm,
                      acc_out, cnt_out,
                      amax_v, src_v, dst_v, att_v, ex_v, rows_v, cnt_rows,
                      idx2, zrow, zcnt, acc_sh, cnt_sh, sem):
    cid = lax.axis_index("c")
    sid = lax.axis_index("s")
    wid = sid * NC + cid
    base0 = wid * CH

    pltpu.sync_copy(amax_hbm, amax_v)

    @pl.loop(0, ZB)
    def _z(i):
        for q in range(D // L):
            zrow[i, pl.ds(q * L, L)] = jnp.zeros((L,), jnp.float32)
        zcnt[i, :] = jnp.zeros((L,), jnp.float32)

    stripe = NPAD // NS
    rb = sid * stripe

    @pl.loop(0, stripe // ZB)
    def _zs(t):
        pltpu.sync_copy(zrow, acc_sh.at[pl.ds(rb + t * ZB, ZB)])
        pltpu.sync_copy(zcnt, cnt_sh.at[pl.ds(rb + t * ZB, ZB)])

    plsc.subcore_barrier()

    @pl.loop(0, NWIN)
    def _win(w):
        base = base0 + w * W
        pltpu.sync_copy(src_hbm.at[pl.ds(base, W)], src_v)
        pltpu.sync_copy(dst_hbm.at[pl.ds(base, W)], dst_v)
        pltpu.sync_copy(att_hbm.at[pl.ds(base, W)], att_v)
        pltpu.async_copy(h_hbm.at[src_v], rows_v, sem).wait()

        @pl.loop(0, W // L)
        def _blk(k):
            sl = pl.ds(k * L, L)
            d = dst_v[sl]
            am = plsc.load_gather(amax_v, [d])
            ex = jnp.exp(att_v[sl] - am)
            ex_v[sl] = ex
            idx2[k, :] = d
            for j in range(L):
                e = k * L + j
                spl = plsc.load_gather(ex_v, [jnp.full((L,), e, jnp.int32)])
                cnt_rows[e, :] = spl
                for q in range(D // L):
                    qs = pl.ds(q * L, L)
                    rows_v[e, qs] = rows_v[e, qs] * spl
            pltpu.sync_copy(rows_v.at[sl], acc_sh.at[idx2.at[k]], add=True)
            pltpu.sync_copy(cnt_rows.at[sl], cnt_sh.at[idx2.at[k]], add=True)

    plsc.subcore_barrier()

    pltpu.sync_copy(acc_sh.at[pl.ds(rb, stripe)],
                    acc_out.at[cid, pl.ds(rb, stripe)])
    pltpu.sync_copy(cnt_sh.at[pl.ds(rb, stripe)],
                    cnt_out.at[cid, pl.ds(rb, stripe)])


def _final_body(h_ref, acc_ref, cnt_ref, w1_ref, w2_ref, o_ref):
    acc = acc_ref[0] + acc_ref[1]
    cnt = cnt_ref[0] + cnt_ref[1]
    denom = jnp.maximum(cnt[:, 0:1], 1e-16)
    hn = acc / denom
    h = h_ref[...]
    y1 = lax.dot_general(h + hn, w1_ref[...], (((1,), (1,)), ((), ())),
                         preferred_element_type=jnp.float32)
    y2 = lax.dot_general(h * hn, w2_ref[...], (((1,), (1,)), ((), ())),
                         preferred_element_type=jnp.float32)
    o_ref[...] = (jnp.where(y1 > 0, y1, 0.01 * y1)
                  + jnp.where(y2 > 0, y2, 0.01 * y2))


def _final(h, acc_parts, cnt_parts, w1, w2):
    BR = 1000
    G = N // BR
    return pl.pallas_call(
        _final_body,
        grid=(G,),
        in_specs=[
            pl.BlockSpec((BR, D), lambda i: (i, 0)),
            pl.BlockSpec((NC, BR, D), lambda i: (0, i, 0)),
            pl.BlockSpec((NC, BR, L), lambda i: (0, i, 0)),
            pl.BlockSpec((D, D), lambda i: (0, 0)),
            pl.BlockSpec((D, D), lambda i: (0, 0)),
        ],
        out_specs=pl.BlockSpec((BR, D), lambda i: (i, 0)),
        out_shape=jax.ShapeDtypeStruct((N, D), jnp.float32),
    )(h, acc_parts, cnt_parts, w1, w2)


def kernel(nfeat, efeat, edge_index, edge_type, relation_weight,
           W_res, W_res_2):
    src = edge_index[0]
    dst = edge_index[1]
    all_proj = _all_proj(nfeat, relation_weight)
    att = _att_kernel(all_proj, efeat, src, dst, edge_type)
    parts = _amax_parts_kernel(att, dst)
    amax = _amax_combine_kernel(parts)
    acc_parts, cnt_parts = _aggregate_kernel(att, dst, src, amax, nfeat)
    return _final(nfeat, acc_parts[:, :N], cnt_parts[:, :N],
                  W_res, W_res_2)


# speedup vs baseline: 8.2110x; 8.2110x over previous
import dataclasses
import functools

import jax
import jax.numpy as jnp
from jax import lax
from jax.experimental import pallas as pl
from jax.experimental.pallas import tpu as pltpu
from jax.experimental.pallas import tpu_sc as plsc

N = 10000
E = 320000
D = 128
R = 8
NC = 2
NS = 16
NW = NC * NS
L = 16
NPAD = 10240
SEG = NPAD // NW
CH = E // NW
W = 80
NWIN = CH // W
NEG = -3.0e38

_mesh = plsc.VectorSubcoreMesh(core_axis_name="c", subcore_axis_name="s")

_sc_params = pltpu.CompilerParams()
if "needs_layout_passes" in pltpu.CompilerParams.__dataclass_fields__:
    _sc_params = dataclasses.replace(_sc_params, needs_layout_passes=False)


def _wid():
    return lax.axis_index("s") * NC + lax.axis_index("c")


def _proj_body(h_ref, w_ref, o_ref):
    o_ref[...] = jnp.dot(h_ref[...], w_ref[0],
                         preferred_element_type=jnp.float32)


def _all_proj(h, rw):
    BR = 1000
    G = N // BR
    return pl.pallas_call(
        _proj_body,
        grid=(R, G),
        in_specs=[
            pl.BlockSpec((BR, D), lambda r, i: (i, 0)),
            pl.BlockSpec((1, D, D), lambda r, i: (r, 0, 0)),
        ],
        out_specs=pl.BlockSpec((BR, D), lambda r, i: (r * G + i, 0)),
        out_shape=jax.ShapeDtypeStruct((R * N, D), jnp.float32),
    )(h, rw)


@functools.partial(
    pl.kernel,
    out_type=jax.ShapeDtypeStruct((E,), jnp.float32),
    mesh=_mesh,
    compiler_params=_sc_params,
    scratch_types=[
        pltpu.VMEM((W,), jnp.int32),
        pltpu.VMEM((W,), jnp.int32),
        pltpu.VMEM((W,), jnp.int32),
        pltpu.VMEM((W,), jnp.int32),
        pltpu.VMEM((W,), jnp.int32),
        pltpu.VMEM((W, D), jnp.float32),
        pltpu.VMEM((W, D), jnp.float32),
        pltpu.VMEM((W, D), jnp.float32),
        pltpu.VMEM((W,), jnp.float32),
        pltpu.SemaphoreType.DMA,
    ],
)
def _att_kernel(ap_hbm, ef_hbm, src_hbm, dst_hbm, et_hbm, att_hbm,
                src_v, dst_v, et_v, fsrc_v, fdst_v, t_v, hr_v, ef_v,
                att_v, sem):
    wid = _wid()
    base0 = wid * CH
    iota = lax.iota(jnp.int32, L)

    @pl.loop(0, NWIN)
    def _win(w):
        base = base0 + w * W
        pltpu.sync_copy(src_hbm.at[pl.ds(base, W)], src_v)
        pltpu.sync_copy(dst_hbm.at[pl.ds(base, W)], dst_v)
        pltpu.sync_copy(et_hbm.at[pl.ds(base, W)], et_v)

        @pl.loop(0, W // L)
        def _idx(k):
            sl = pl.ds(k * L, L)
            et = et_v[sl]
            fsrc_v[sl] = et * N + src_v[sl]
            fdst_v[sl] = et * N + dst_v[sl]

        cp1 = pltpu.async_copy(ap_hbm.at[fsrc_v], t_v, sem)
        cp2 = pltpu.async_copy(ap_hbm.at[fdst_v], hr_v, sem)
        cp3 = pltpu.async_copy(ef_hbm.at[pl.ds(base, W)], ef_v, sem)
        cp1.wait()
        cp2.wait()
        cp3.wait()

        @pl.loop(0, W // L)
        def _blk(k):
            acc = jnp.zeros((L,), jnp.float32)
            for j in range(L):
                e = k * L + j
                p = jnp.zeros((L,), jnp.float32)
                for q in range(D // L):
                    sl = pl.ds(q * L, L)
                    t = t_v[e, sl]
                    x = hr_v[e, sl] + ef_v[e, sl]
                    th = 1.0 - 2.0 / (jnp.exp(2.0 * x) + 1.0)
                    p = p + t * th
                s = jnp.sum(p)
                acc = jnp.where(iota == j, s, acc)
            att_v[pl.ds(k * L, L)] = acc

        pltpu.sync_copy(att_v, att_hbm.at[pl.ds(base, W)])


@functools.partial(
    pl.kernel,
    out_type=jax.ShapeDtypeStruct((NW * NPAD,), jnp.float32),
    mesh=_mesh,
    compiler_params=_sc_params,
    scratch_types=[
        pltpu.VMEM((NPAD,), jnp.float32),
        pltpu.VMEM((W,), jnp.float32),
        pltpu.VMEM((W,), jnp.int32),
        pltpu.SemaphoreType.DMA,
    ],
)
def _amax_parts_kernel(att_hbm, dst_hbm, out_hbm, tab_v, att_v, dst_v, sem):
    wid = _wid()
    base0 = wid * CH

    @pl.loop(0, NPAD // L)
    def _init(i):
        tab_v[pl.ds(i * L, L)] = jnp.full((L,), NEG, jnp.float32)

    @pl.loop(0, NWIN)
    def _win(w):
        base = base0 + w * W
        pltpu.sync_copy(att_hbm.at[pl.ds(base, W)], att_v)
        pltpu.sync_copy(dst_hbm.at[pl.ds(base, W)], dst_v)

        @pl.loop(0, W // L)
        def _blk(k):
            sl = pl.ds(k * L, L)
            d = dst_v[sl]
            a = att_v[sl]

            def body(_):
                cur = plsc.load_gather(tab_v, [d])
                need = a > cur
                plsc.store_scatter(tab_v, [d], a, mask=need)
                return jnp.any(need)

            lax.while_loop(lambda go: go, body, jnp.bool_(True))

    pltpu.sync_copy(tab_v, out_hbm.at[pl.ds(wid * NPAD, NPAD)])


@functools.partial(
    pl.kernel,
    out_type=jax.ShapeDtypeStruct((NPAD,), jnp.float32),
    mesh=_mesh,
    compiler_params=_sc_params,
    scratch_types=[
        pltpu.VMEM((NW * SEG,), jnp.float32),
        pltpu.VMEM((SEG,), jnp.float32),
        pltpu.SemaphoreType.DMA,
    ],
)
def _amax_combine_kernel(parts_hbm, out_hbm, buf_v, res_v, sem):
    wid = _wid()
    col = wid * SEG
    for k in range(NW):
        pltpu.sync_copy(parts_hbm.at[pl.ds(k * NPAD + col, SEG)],
                        buf_v.at[pl.ds(k * SEG, SEG)])

    @pl.loop(0, SEG // L)
    def _blk(t):
        m = buf_v[pl.ds(t * L, L)]
        for k in range(1, NW):
            m = jnp.maximum(m, buf_v[pl.ds(k * SEG + t * L, L)])
        res_v[pl.ds(t * L, L)] = jnp.where(m > -1.0e38, m, 0.0)

    pltpu.sync_copy(res_v, out_hbm.at[pl.ds(col, SEG)])


@functools.partial(
    pl.kernel,
    out_type=(
        jax.ShapeDtypeStruct((E, L), jnp.float32),
        jax.ShapeDtypeStruct((NW * NPAD,), jnp.float32),
    ),
    mesh=_mesh,
    compiler_params=_sc_params,
    scratch_types=[
        pltpu.VMEM((NPAD,), jnp.float32),
        pltpu.VMEM((NPAD,), jnp.float32),
        pltpu.VMEM((W,), jnp.float32),
        pltpu.VMEM((W,), jnp.int32),
        pltpu.VMEM((W,), jnp.float32),
        pltpu.VMEM((W, L), jnp.float32),
        pltpu.VMEM((L,), jnp.int32),
        pltpu.VMEM((L,), jnp.float32),
        pltpu.SemaphoreType.DMA,
    ],
)
def _ex_rows_kernel(att_hbm, dst_hbm, amax_hbm, out_hbm, dparts_hbm,
                    amax_v, dtab_v, att_v, dst_v, exl_v, ex16_v,
                    kbuf, sbuf, sem):
    wid = _wid()
    base0 = wid * CH
    iota = lax.iota(jnp.int32, L)
    pltpu.sync_copy(amax_hbm, amax_v)

    @pl.loop(0, NPAD // L)
    def _init(i):
        dtab_v[pl.ds(i * L, L)] = jnp.zeros((L,), jnp.float32)

    @pl.loop(0, NWIN)
    def _win(w):
        base = base0 + w * W
        pltpu.sync_copy(att_hbm.at[pl.ds(base, W)], att_v)
        pltpu.sync_copy(dst_hbm.at[pl.ds(base, W)], dst_v)

        @pl.loop(0, W // L)
        def _blk(k):
            sl = pl.ds(k * L, L)
            d = dst_v[sl]
            am = plsc.load_gather(amax_v, [d])
            ex = jnp.exp(att_v[sl] - am)
            exl_v[sl] = ex
            sk, sv = plsc.sort_key_val(d, ex)
            for sh in (1, 2, 4, 8):
                kbuf[...] = sk
                sbuf[...] = sv
                pidx = jnp.maximum(iota - sh, 0)
                pk = plsc.load_gather(kbuf, [pidx])
                pv = plsc.load_gather(sbuf, [pidx])
                take = jnp.logical_and(iota >= sh, pk == sk)
                sv = sv + jnp.where(take, pv, 0.0)
            kbuf[...] = sk
            nk = plsc.load_gather(kbuf, [jnp.minimum(iota + 1, L - 1)])
            islast = jnp.logical_or(nk != sk, iota == L - 1)
            plsc.addupdate_scatter(dtab_v, [sk], sv, mask=islast)

        @pl.loop(0, W // L)
        def _spl(k):
            for j in range(L):
                e = k * L + j
                ex16_v[e, :] = plsc.load_gather(
                    exl_v, [jnp.full((L,), e, jnp.int32)])

        pltpu.sync_copy(ex16_v, out_hbm.at[pl.ds(base, W)])

    pltpu.sync_copy(dtab_v, dparts_hbm.at[pl.ds(wid * NPAD, NPAD)])


@functools.partial(
    pl.kernel,
    out_type=jax.ShapeDtypeStruct((NPAD,), jnp.float32),
    mesh=_mesh,
    compiler_params=_sc_params,
    scratch_types=[
        pltpu.VMEM((NW * SEG,), jnp.float32),
        pltpu.VMEM((SEG,), jnp.float32),
        pltpu.SemaphoreType.DMA,
    ],
)
def _denom_combine_kernel(parts_hbm, out_hbm, buf_v, res_v, sem):
    wid = _wid()
    col = wid * SEG
    for k in range(NW):
        pltpu.sync_copy(parts_hbm.at[pl.ds(k * NPAD + col, SEG)],
                        buf_v.at[pl.ds(k * SEG, SEG)])

    @pl.loop(0, SEG // L)
    def _blk(t):
        m = buf_v[pl.ds(t * L, L)]
        for k in range(1, NW):
            m = m + buf_v[pl.ds(k * SEG + t * L, L)]
        res_v[pl.ds(t * L, L)] = m

    pltpu.sync_copy(res_v, out_hbm.at[pl.ds(col, SEG)])


@functools.partial(
    pl.kernel,
    out_type=jax.ShapeDtypeStruct((NC * NPAD, D), jnp.float32),
    mesh=_mesh,
    compiler_params=_sc_params,
    scratch_types=[
        pltpu.VMEM((W,), jnp.int32),
        pltpu.VMEM((W,), jnp.int32),
        pltpu.VMEM((W, L), jnp.float32),
        pltpu.VMEM((W, D), jnp.float32),
        [pltpu.VMEM((L,), jnp.int32) for _ in range(W // L)],
        pltpu.VMEM_SHARED((NPAD, D), jnp.float32),
        pltpu.SemaphoreType.DMA,
    ],
)
def _aggregate_kernel(ex16_hbm, dst_hbm, src_hbm, h_hbm,
                      acc_out,
                      src_v, dst_v, ex_v, rows_v,
                      idx2, acc_sh, sem):
    cid = lax.axis_index("c")
    sid = lax.axis_index("s")
    wid = sid * NC + cid
    base0 = wid * CH

    @pl.loop(0, W)
    def _z(i):
        for q in range(D // L):
            rows_v[i, pl.ds(q * L, L)] = jnp.zeros((L,), jnp.float32)

    stripe = NPAD // NS
    rb = sid * stripe

    @pl.loop(0, stripe // W)
    def _zs(t):
        pltpu.sync_copy(rows_v, acc_sh.at[pl.ds(rb + t * W, W)])

    plsc.subcore_barrier()

    @pl.loop(0, NWIN)
    def _win(w):
        base = base0 + w * W
        pltpu.sync_copy(src_hbm.at[pl.ds(base, W)], src_v)
        pltpu.sync_copy(dst_hbm.at[pl.ds(base, W)], dst_v)
        cp1 = pltpu.async_copy(ex16_hbm.at[pl.ds(base, W)], ex_v, sem)
        cp2 = pltpu.async_copy(h_hbm.at[src_v], rows_v, sem)
        cp1.wait()
        cp2.wait()

        for k in range(W // L):
            sl = pl.ds(k * L, L)
            idx2[k][...] = dst_v[sl]
            for j in range(L):
                e = k * L + j
                exr = ex_v[e, :]
                for q in range(D // L):
                    qs = pl.ds(q * L, L)
                    rows_v[e, qs] = rows_v[e, qs] * exr
            pltpu.sync_copy(rows_v.at[sl], acc_sh.at[idx2[k]], add=True)

    plsc.subcore_barrier()

    ob = cid * NPAD + rb
    pltpu.sync_copy(acc_sh.at[pl.ds(rb, stripe)],
                    acc_out.at[pl.ds(ob, stripe)])


def _final_body(h_ref, acc_ref, den_ref, w1_ref, w2_ref, o_ref):
    acc = acc_ref[0] + acc_ref[1]
    denom = jnp.maximum(den_ref[...], 1e-16)
    hn = acc / denom
    h = h_ref[...]
    y1 = lax.dot_general(h + hn, w1_ref[...], (((1,), (1,)), ((), ())),
                         preferred_element_type=jnp.float32)
    y2 = lax.dot_general(h * hn, w2_ref[...], (((1,), (1,)), ((), ())),
                         preferred_element_type=jnp.float32)
    o_ref[...] = (jnp.where(y1 > 0, y1, 0.01 * y1)
                  + jnp.where(y2 > 0, y2, 0.01 * y2))


def _final(h, acc_parts, denom, w1, w2):
    BR = 1000
    G = N // BR
    return pl.pallas_call(
        _final_body,
        grid=(G,),
        in_specs=[
            pl.BlockSpec((BR, D), lambda i: (i, 0)),
            pl.BlockSpec((NC, BR, D), lambda i: (0, i, 0)),
            pl.BlockSpec((BR, 1), lambda i: (i, 0)),
            pl.BlockSpec((D, D), lambda i: (0, 0)),
            pl.BlockSpec((D, D), lambda i: (0, 0)),
        ],
        out_specs=pl.BlockSpec((BR, D), lambda i: (i, 0)),
        out_shape=jax.ShapeDtypeStruct((N, D), jnp.float32),
    )(h, acc_parts, denom, w1, w2)


def kernel(nfeat, efeat, edge_index, edge_type, relation_weight,
           W_res, W_res_2):
    src = edge_index[0]
    dst = edge_index[1]
    all_proj = _all_proj(nfeat, relation_weight)
    att = _att_kernel(all_proj, efeat, src, dst, edge_type)
    parts = _amax_parts_kernel(att, dst)
    amax = _amax_combine_kernel(parts)
    ex16, dparts = _ex_rows_kernel(att, dst, amax)
    denom = _denom_combine_kernel(dparts)
    acc_flat = _aggregate_kernel(ex16, dst, src, nfeat)
    acc_parts = acc_flat.reshape(NC, NPAD, D)[:, :N]
    return _final(nfeat, acc_parts, denom.reshape(NPAD, 1)[:N],
                  W_res, W_res_2)


# speedup vs baseline: 8.2622x; 1.0062x over previous
import dataclasses
import functools

import jax
import jax.numpy as jnp
from jax import lax
from jax.experimental import pallas as pl
from jax.experimental.pallas import tpu as pltpu
from jax.experimental.pallas import tpu_sc as plsc

N = 10000
E = 320000
D = 128
R = 8
NC = 2
NS = 16
NW = NC * NS
L = 16
NPAD = 10240
SEG = NPAD // NW
CH = E // NW
W = 80
NWIN = CH // W
NEG = -3.0e38

_mesh = plsc.VectorSubcoreMesh(core_axis_name="c", subcore_axis_name="s")

_sc_params = pltpu.CompilerParams()
if "needs_layout_passes" in pltpu.CompilerParams.__dataclass_fields__:
    _sc_params = dataclasses.replace(_sc_params, needs_layout_passes=False)


def _wid():
    return lax.axis_index("s") * NC + lax.axis_index("c")


def _proj_body(h_ref, w_ref, o_ref):
    o_ref[...] = jnp.dot(h_ref[...], w_ref[0],
                         preferred_element_type=jnp.float32)


def _all_proj(h, rw):
    BR = 1000
    G = N // BR
    return pl.pallas_call(
        _proj_body,
        grid=(R, G),
        in_specs=[
            pl.BlockSpec((BR, D), lambda r, i: (i, 0)),
            pl.BlockSpec((1, D, D), lambda r, i: (r, 0, 0)),
        ],
        out_specs=pl.BlockSpec((BR, D), lambda r, i: (r * G + i, 0)),
        out_shape=jax.ShapeDtypeStruct((R * N, D), jnp.float32),
    )(h, rw)


def _edge_bufs():
    return [
        pltpu.VMEM((W,), jnp.int32),
        pltpu.VMEM((W,), jnp.int32),
        pltpu.VMEM((W,), jnp.int32),
        pltpu.VMEM((W,), jnp.int32),
        pltpu.VMEM((W,), jnp.int32),
        pltpu.VMEM((W, D), jnp.float32),
        pltpu.VMEM((W, D), jnp.float32),
        pltpu.VMEM((W, D), jnp.float32),
        pltpu.VMEM((W,), jnp.float32),
    ]


@functools.partial(
    pl.kernel,
    out_type=(
        jax.ShapeDtypeStruct((E,), jnp.float32),
        jax.ShapeDtypeStruct((NW * NPAD,), jnp.float32),
    ),
    mesh=_mesh,
    compiler_params=_sc_params,
    scratch_types=(
        _edge_bufs() + _edge_bufs()
        + [pltpu.VMEM((NPAD,), jnp.float32),
           pltpu.SemaphoreType.DMA,
           pltpu.SemaphoreType.DMA]
    ),
)
def _att_kernel(ap_hbm, ef_hbm, src_hbm, dst_hbm, et_hbm,
                att_hbm, mparts_hbm,
                srcA, dstA, etA, fsA, fdA, tA, hA, eA, attA,
                srcB, dstB, etB, fsB, fdB, tB, hB, eB, attB,
                tab_v, semA, semB):
    wid = _wid()
    base0 = wid * CH
    iota = lax.iota(jnp.int32, L)

    @pl.loop(0, NPAD // L)
    def _init(i):
        tab_v[pl.ds(i * L, L)] = jnp.full((L,), NEG, jnp.float32)

    def idx_issue(sb, db, eb, base, sem):
        pltpu.async_copy(src_hbm.at[pl.ds(base, W)], sb, sem)
        pltpu.async_copy(dst_hbm.at[pl.ds(base, W)], db, sem)
        pltpu.async_copy(et_hbm.at[pl.ds(base, W)], eb, sem)

    def wait_idx(sb, db, eb, sem):
        pltpu.make_async_copy(src_hbm.at[pl.ds(0, W)], sb, sem).wait()
        pltpu.make_async_copy(src_hbm.at[pl.ds(0, W)], db, sem).wait()
        pltpu.make_async_copy(src_hbm.at[pl.ds(0, W)], eb, sem).wait()

    def flat_gather(sb, db, eb, fs, fd, tb, hb, ebuf, base, sem):
        @pl.loop(0, W // L)
        def _idx(k):
            sl = pl.ds(k * L, L)
            e_ = eb[sl]
            fs[sl] = e_ * N + sb[sl]
            fd[sl] = e_ * N + db[sl]
        pltpu.async_copy(ap_hbm.at[fs], tb, sem)
        pltpu.async_copy(ap_hbm.at[fd], hb, sem)
        pltpu.async_copy(ef_hbm.at[pl.ds(base, W)], ebuf, sem)

    def wait_gathers(tb, hb, ebuf, sem):
        pltpu.make_async_copy(ef_hbm.at[pl.ds(0, W)], tb, sem).wait()
        pltpu.make_async_copy(ef_hbm.at[pl.ds(0, W)], hb, sem).wait()
        pltpu.make_async_copy(ef_hbm.at[pl.ds(0, W)], ebuf, sem).wait()

    def compute(db, tb, hb, ebuf, attb):
        @pl.loop(0, W // L)
        def _blk(k):
            acc = jnp.zeros((L,), jnp.float32)
            for j in range(L):
                e = k * L + j
                p = jnp.zeros((L,), jnp.float32)
                for q in range(D // L):
                    sl = pl.ds(q * L, L)
                    t = tb[e, sl]
                    x = hb[e, sl] + ebuf[e, sl]
                    th = 1.0 - 2.0 / (jnp.exp(2.0 * x) + 1.0)
                    p = p + t * th
                s = jnp.sum(p)
                acc = jnp.where(iota == j, s, acc)
            attb[pl.ds(k * L, L)] = acc
            d = db[pl.ds(k * L, L)]

            def body(_):
                cur = plsc.load_gather(tab_v, [d])
                need = acc > cur
                plsc.store_scatter(tab_v, [d], acc, mask=need)
                return jnp.any(need)

            lax.while_loop(lambda go: go, body, jnp.bool_(True))

    idx_issue(srcA, dstA, etA, base0, semA)
    wait_idx(srcA, dstA, etA, semA)
    flat_gather(srcA, dstA, etA, fsA, fdA, tA, hA, eA, base0, semA)
    idx_issue(srcB, dstB, etB, base0 + W, semB)
    wait_idx(srcB, dstB, etB, semB)
    flat_gather(srcB, dstB, etB, fsB, fdB, tB, hB, eB, base0 + W, semB)
    wait_gathers(tA, hA, eA, semA)
    compute(dstA, tA, hA, eA, attA)
    pltpu.async_copy(attA, att_hbm.at[pl.ds(base0, W)], semA)
    pltpu.make_async_copy(attA, att_hbm.at[pl.ds(base0, W)], semA).wait()

    @pl.loop(0, (NWIN - 1) // 2)
    def _body(i):
        wb = 1 + 2 * i
        base_b = base0 + wb * W
        base_a = base_b + W
        base_p = jnp.minimum(base_a + W, E - W)

        idx_issue(srcA, dstA, etA, base_a, semA)
        wait_gathers(tB, hB, eB, semB)
        compute(dstB, tB, hB, eB, attB)
        pltpu.async_copy(attB, att_hbm.at[pl.ds(base_b, W)], semB)
        wait_idx(srcA, dstA, etA, semA)
        flat_gather(srcA, dstA, etA, fsA, fdA, tA, hA, eA, base_a, semA)
        pltpu.make_async_copy(attB, att_hbm.at[pl.ds(base_b, W)], semB).wait()

        idx_issue(srcB, dstB, etB, base_p, semB)
        wait_gathers(tA, hA, eA, semA)
        compute(dstA, tA, hA, eA, attA)
        pltpu.async_copy(attA, att_hbm.at[pl.ds(base_a, W)], semA)
        wait_idx(srcB, dstB, etB, semB)
        flat_gather(srcB, dstB, etB, fsB, fdB, tB, hB, eB, base_p, semB)
        pltpu.make_async_copy(attA, att_hbm.at[pl.ds(base_a, W)], semA).wait()

    wait_gathers(tB, hB, eB, semB)

    pltpu.sync_copy(tab_v, mparts_hbm.at[pl.ds(wid * NPAD, NPAD)])


@functools.partial(
    pl.kernel,
    out_type=jax.ShapeDtypeStruct((NPAD,), jnp.float32),
    mesh=_mesh,
    compiler_params=_sc_params,
    scratch_types=[
        pltpu.VMEM((NW * SEG,), jnp.float32),
        pltpu.VMEM((SEG,), jnp.float32),
        pltpu.SemaphoreType.DMA,
    ],
)
def _amax_combine_kernel(parts_hbm, out_hbm, buf_v, res_v, sem):
    wid = _wid()
    col = wid * SEG
    for k in range(NW):
        pltpu.sync_copy(parts_hbm.at[pl.ds(k * NPAD + col, SEG)],
                        buf_v.at[pl.ds(k * SEG, SEG)])

    @pl.loop(0, SEG // L)
    def _blk(t):
        m = buf_v[pl.ds(t * L, L)]
        for k in range(1, NW):
            m = jnp.maximum(m, buf_v[pl.ds(k * SEG + t * L, L)])
        res_v[pl.ds(t * L, L)] = jnp.where(m > -1.0e38, m, 0.0)

    pltpu.sync_copy(res_v, out_hbm.at[pl.ds(col, SEG)])


@functools.partial(
    pl.kernel,
    out_type=(
        jax.ShapeDtypeStruct((E, L), jnp.float32),
        jax.ShapeDtypeStruct((NW * NPAD,), jnp.float32),
    ),
    mesh=_mesh,
    compiler_params=_sc_params,
    scratch_types=[
        pltpu.VMEM((NPAD,), jnp.float32),
        pltpu.VMEM((NPAD,), jnp.float32),
        pltpu.VMEM((W,), jnp.float32),
        pltpu.VMEM((W,), jnp.int32),
        pltpu.VMEM((W,), jnp.float32),
        pltpu.VMEM((W, L), jnp.float32),
        pltpu.VMEM((L,), jnp.int32),
        pltpu.VMEM((L,), jnp.float32),
        pltpu.SemaphoreType.DMA,
    ],
)
def _ex_rows_kernel(att_hbm, dst_hbm, amax_hbm, out_hbm, dparts_hbm,
                    amax_v, dtab_v, att_v, dst_v, exl_v, ex16_v,
                    kbuf, sbuf, sem):
    wid = _wid()
    base0 = wid * CH
    iota = lax.iota(jnp.int32, L)
    pltpu.sync_copy(amax_hbm, amax_v)

    @pl.loop(0, NPAD // L)
    def _init(i):
        dtab_v[pl.ds(i * L, L)] = jnp.zeros((L,), jnp.float32)

    @pl.loop(0, NWIN)
    def _win(w):
        base = base0 + w * W
        pltpu.sync_copy(att_hbm.at[pl.ds(base, W)], att_v)
        pltpu.sync_copy(dst_hbm.at[pl.ds(base, W)], dst_v)

        @pl.loop(0, W // L)
        def _blk(k):
            sl = pl.ds(k * L, L)
            d = dst_v[sl]
            am = plsc.load_gather(amax_v, [d])
            ex = jnp.exp(att_v[sl] - am)
            exl_v[sl] = ex
            sk, sv = plsc.sort_key_val(d, ex)
            for sh in (1, 2, 4, 8):
                kbuf[...] = sk
                sbuf[...] = sv
                pidx = jnp.maximum(iota - sh, 0)
                pk = plsc.load_gather(kbuf, [pidx])
                pv = plsc.load_gather(sbuf, [pidx])
                take = jnp.logical_and(iota >= sh, pk == sk)
                sv = sv + jnp.where(take, pv, 0.0)
            kbuf[...] = sk
            nk = plsc.load_gather(kbuf, [jnp.minimum(iota + 1, L - 1)])
            islast = jnp.logical_or(nk != sk, iota == L - 1)
            plsc.addupdate_scatter(dtab_v, [sk], sv, mask=islast)

        @pl.loop(0, W // L)
        def _spl(k):
            for j in range(L):
                e = k * L + j
                ex16_v[e, :] = plsc.load_gather(
                    exl_v, [jnp.full((L,), e, jnp.int32)])

        pltpu.sync_copy(ex16_v, out_hbm.at[pl.ds(base, W)])

    pltpu.sync_copy(dtab_v, dparts_hbm.at[pl.ds(wid * NPAD, NPAD)])


@functools.partial(
    pl.kernel,
    out_type=jax.ShapeDtypeStruct((NPAD,), jnp.float32),
    mesh=_mesh,
    compiler_params=_sc_params,
    scratch_types=[
        pltpu.VMEM((NW * SEG,), jnp.float32),
        pltpu.VMEM((SEG,), jnp.float32),
        pltpu.SemaphoreType.DMA,
    ],
)
def _denom_combine_kernel(parts_hbm, out_hbm, buf_v, res_v, sem):
    wid = _wid()
    col = wid * SEG
    for k in range(NW):
        pltpu.sync_copy(parts_hbm.at[pl.ds(k * NPAD + col, SEG)],
                        buf_v.at[pl.ds(k * SEG, SEG)])

    @pl.loop(0, SEG // L)
    def _blk(t):
        m = buf_v[pl.ds(t * L, L)]
        for k in range(1, NW):
            m = m + buf_v[pl.ds(k * SEG + t * L, L)]
        res_v[pl.ds(t * L, L)] = m

    pltpu.sync_copy(res_v, out_hbm.at[pl.ds(col, SEG)])


@functools.partial(
    pl.kernel,
    out_type=jax.ShapeDtypeStruct((NC * NPAD, D), jnp.float32),
    mesh=_mesh,
    compiler_params=_sc_params,
    scratch_types=[
        pltpu.VMEM((W,), jnp.int32),
        pltpu.VMEM((W,), jnp.int32),
        pltpu.VMEM((W, L), jnp.float32),
        pltpu.VMEM((W, D), jnp.float32),
        [pltpu.VMEM((L,), jnp.int32) for _ in range(W // L)],
        pltpu.VMEM_SHARED((NPAD, D), jnp.float32),
        pltpu.SemaphoreType.DMA,
    ],
)
def _aggregate_kernel(ex16_hbm, dst_hbm, src_hbm, h_hbm,
                      acc_out,
                      src_v, dst_v, ex_v, rows_v,
                      idx2, acc_sh, sem):
    cid = lax.axis_index("c")
    sid = lax.axis_index("s")
    wid = sid * NC + cid
    base0 = wid * CH

    @pl.loop(0, W)
    def _z(i):
        for q in range(D // L):
            rows_v[i, pl.ds(q * L, L)] = jnp.zeros((L,), jnp.float32)

    stripe = NPAD // NS
    rb = sid * stripe

    @pl.loop(0, stripe // W)
    def _zs(t):
        pltpu.sync_copy(rows_v, acc_sh.at[pl.ds(rb + t * W, W)])

    plsc.subcore_barrier()

    @pl.loop(0, NWIN)
    def _win(w):
        base = base0 + w * W
        pltpu.sync_copy(src_hbm.at[pl.ds(base, W)], src_v)
        pltpu.sync_copy(dst_hbm.at[pl.ds(base, W)], dst_v)
        cp1 = pltpu.async_copy(ex16_hbm.at[pl.ds(base, W)], ex_v, sem)
        cp2 = pltpu.async_copy(h_hbm.at[src_v], rows_v, sem)
        cp1.wait()
        cp2.wait()

        for k in range(W // L):
            sl = pl.ds(k * L, L)
            idx2[k][...] = dst_v[sl]
            for j in range(L):
                e = k * L + j
                exr = ex_v[e, :]
                for q in range(D // L):
                    qs = pl.ds(q * L, L)
                    rows_v[e, qs] = rows_v[e, qs] * exr
            pltpu.sync_copy(rows_v.at[sl], acc_sh.at[idx2[k]], add=True)

    plsc.subcore_barrier()

    ob = cid * NPAD + rb
    pltpu.sync_copy(acc_sh.at[pl.ds(rb, stripe)],
                    acc_out.at[pl.ds(ob, stripe)])


def _final_body(h_ref, acc_ref, den_ref, w1_ref, w2_ref, o_ref):
    acc = acc_ref[0] + acc_ref[1]
    denom = jnp.maximum(den_ref[...], 1e-16)
    hn = acc / denom
    h = h_ref[...]
    y1 = lax.dot_general(h + hn, w1_ref[...], (((1,), (1,)), ((), ())),
                         preferred_element_type=jnp.float32)
    y2 = lax.dot_general(h * hn, w2_ref[...], (((1,), (1,)), ((), ())),
                         preferred_element_type=jnp.float32)
    o_ref[...] = (jnp.where(y1 > 0, y1, 0.01 * y1)
                  + jnp.where(y2 > 0, y2, 0.01 * y2))


def _final(h, acc_parts, denom, w1, w2):
    BR = 1000
    G = N // BR
    return pl.pallas_call(
        _final_body,
        grid=(G,),
        in_specs=[
            pl.BlockSpec((BR, D), lambda i: (i, 0)),
            pl.BlockSpec((NC, BR, D), lambda i: (0, i, 0)),
            pl.BlockSpec((BR, 1), lambda i: (i, 0)),
            pl.BlockSpec((D, D), lambda i: (0, 0)),
            pl.BlockSpec((D, D), lambda i: (0, 0)),
        ],
        out_specs=pl.BlockSpec((BR, D), lambda i: (i, 0)),
        out_shape=jax.ShapeDtypeStruct((N, D), jnp.float32),
    )(h, acc_parts, denom, w1, w2)


def kernel(nfeat, efeat, edge_index, edge_type, relation_weight,
           W_res, W_res_2):
    src = edge_index[0]
    dst = edge_index[1]
    all_proj = _all_proj(nfeat, relation_weight)
    att, parts = _att_kernel(all_proj, efeat, src, dst, edge_type)
    amax = _amax_combine_kernel(parts)
    ex16, dparts = _ex_rows_kernel(att, dst, amax)
    denom = _denom_combine_kernel(dparts)
    acc_flat = _aggregate_kernel(ex16, dst, src, nfeat)
    acc_parts = acc_flat.reshape(NC, NPAD, D)[:, :N]
    return _final(nfeat, acc_parts, denom.reshape(NPAD, 1)[:N],
                  W_res, W_res_2)


# speedup vs baseline: 15.7697x; 1.9087x over previous
import dataclasses
import functools

import jax
import jax.numpy as jnp
from jax import lax
from jax.experimental import pallas as pl
from jax.experimental.pallas import tpu as pltpu
from jax.experimental.pallas import tpu_sc as plsc

N = 10000
E = 320000
D = 128
R = 8
NC = 2
NS = 16
NW = NC * NS
L = 16
NPAD = 10240
SEG = NPAD // NW
CH = E // NW
W = 80
NWIN = CH // W
NEG = -3.0e38

_mesh = plsc.VectorSubcoreMesh(core_axis_name="c", subcore_axis_name="s")

_sc_params = pltpu.CompilerParams()
if "needs_layout_passes" in pltpu.CompilerParams.__dataclass_fields__:
    _sc_params = dataclasses.replace(_sc_params, needs_layout_passes=False)


def _wid():
    return lax.axis_index("s") * NC + lax.axis_index("c")


def _proj_body(h_ref, w_ref, o_ref):
    o_ref[...] = jnp.dot(h_ref[...], w_ref[0],
                         preferred_element_type=jnp.float32)


def _all_proj(h, rw):
    BR = 1000
    G = N // BR
    return pl.pallas_call(
        _proj_body,
        grid=(R, G),
        in_specs=[
            pl.BlockSpec((BR, D), lambda r, i: (i, 0)),
            pl.BlockSpec((1, D, D), lambda r, i: (r, 0, 0)),
        ],
        out_specs=pl.BlockSpec((BR, D), lambda r, i: (r * G + i, 0)),
        out_shape=jax.ShapeDtypeStruct((R * N, D), jnp.float32),
    )(h, rw)


def _edge_bufs():
    return [
        pltpu.VMEM((W,), jnp.int32),
        pltpu.VMEM((W,), jnp.int32),
        pltpu.VMEM((W,), jnp.int32),
        pltpu.VMEM((W,), jnp.int32),
        pltpu.VMEM((W,), jnp.int32),
        pltpu.VMEM((W, D), jnp.float32),
        pltpu.VMEM((W, D), jnp.float32),
        pltpu.VMEM((W, D), jnp.float32),
        pltpu.VMEM((W,), jnp.float32),
    ]


@functools.partial(
    pl.kernel,
    out_type=(
        jax.ShapeDtypeStruct((E,), jnp.float32),
        jax.ShapeDtypeStruct((NW * NPAD,), jnp.float32),
    ),
    mesh=_mesh,
    compiler_params=_sc_params,
    scratch_types=(
        _edge_bufs() + _edge_bufs()
        + [pltpu.VMEM((NPAD,), jnp.float32),
           pltpu.SemaphoreType.DMA,
           pltpu.SemaphoreType.DMA]
    ),
)
def _att_kernel(ap_hbm, ef_hbm, src_hbm, dst_hbm, et_hbm,
                att_hbm, mparts_hbm,
                srcA, dstA, etA, fsA, fdA, tA, hA, eA, attA,
                srcB, dstB, etB, fsB, fdB, tB, hB, eB, attB,
                tab_v, semA, semB):
    wid = _wid()
    base0 = wid * CH
    iota = lax.iota(jnp.int32, L)

    @pl.loop(0, NPAD // L)
    def _init(i):
        tab_v[pl.ds(i * L, L)] = jnp.full((L,), NEG, jnp.float32)

    def idx_issue(sb, db, eb, base, sem):
        pltpu.async_copy(src_hbm.at[pl.ds(base, W)], sb, sem)
        pltpu.async_copy(dst_hbm.at[pl.ds(base, W)], db, sem)
        pltpu.async_copy(et_hbm.at[pl.ds(base, W)], eb, sem)

    def wait_idx(sb, db, eb, sem):
        pltpu.make_async_copy(src_hbm.at[pl.ds(0, W)], sb, sem).wait()
        pltpu.make_async_copy(src_hbm.at[pl.ds(0, W)], db, sem).wait()
        pltpu.make_async_copy(src_hbm.at[pl.ds(0, W)], eb, sem).wait()

    def flat_gather(sb, db, eb, fs, fd, tb, hb, ebuf, base, sem):
        @pl.loop(0, W // L)
        def _idx(k):
            sl = pl.ds(k * L, L)
            e_ = eb[sl]
            fs[sl] = e_ * N + sb[sl]
            fd[sl] = e_ * N + db[sl]
        pltpu.async_copy(ap_hbm.at[fs], tb, sem)
        pltpu.async_copy(ap_hbm.at[fd], hb, sem)
        pltpu.async_copy(ef_hbm.at[pl.ds(base, W)], ebuf, sem)

    def wait_gathers(tb, hb, ebuf, sem):
        pltpu.make_async_copy(ef_hbm.at[pl.ds(0, W)], tb, sem).wait()
        pltpu.make_async_copy(ef_hbm.at[pl.ds(0, W)], hb, sem).wait()
        pltpu.make_async_copy(ef_hbm.at[pl.ds(0, W)], ebuf, sem).wait()

    def compute(db, tb, hb, ebuf, attb):
        @functools.partial(plsc.parallel_loop, 0, W // L, unroll=2)
        def _blk(k):
            acc = jnp.zeros((L,), jnp.float32)
            for j in range(L):
                e = k * L + j
                terms = []
                for q in range(D // L):
                    sl = pl.ds(q * L, L)
                    t = tb[e, sl]
                    x = hb[e, sl] + ebuf[e, sl]
                    th = 1.0 - 2.0 / (jnp.exp(2.0 * x) + 1.0)
                    terms.append(t * th)
                while len(terms) > 1:
                    terms = [a + b for a, b in
                             zip(terms[::2], terms[1::2])]
                s = jnp.sum(terms[0])
                acc = jnp.where(iota == j, s, acc)
            attb[pl.ds(k * L, L)] = acc

        @pl.loop(0, W // L)
        def _mx(k):
            sl = pl.ds(k * L, L)
            a = attb[sl]
            d = db[sl]

            def body(_):
                cur = plsc.load_gather(tab_v, [d])
                need = a > cur
                plsc.store_scatter(tab_v, [d], a, mask=need)
                return jnp.any(need)

            lax.while_loop(lambda go: go, body, jnp.bool_(True))

    idx_issue(srcA, dstA, etA, base0, semA)
    wait_idx(srcA, dstA, etA, semA)
    flat_gather(srcA, dstA, etA, fsA, fdA, tA, hA, eA, base0, semA)
    idx_issue(srcB, dstB, etB, base0 + W, semB)
    wait_idx(srcB, dstB, etB, semB)
    flat_gather(srcB, dstB, etB, fsB, fdB, tB, hB, eB, base0 + W, semB)
    wait_gathers(tA, hA, eA, semA)
    compute(dstA, tA, hA, eA, attA)
    pltpu.async_copy(attA, att_hbm.at[pl.ds(base0, W)], semA)
    pltpu.make_async_copy(attA, att_hbm.at[pl.ds(base0, W)], semA).wait()

    @pl.loop(0, (NWIN - 1) // 2)
    def _body(i):
        wb = 1 + 2 * i
        base_b = base0 + wb * W
        base_a = base_b + W
        base_p = jnp.minimum(base_a + W, E - W)

        idx_issue(srcA, dstA, etA, base_a, semA)
        wait_gathers(tB, hB, eB, semB)
        compute(dstB, tB, hB, eB, attB)
        pltpu.async_copy(attB, att_hbm.at[pl.ds(base_b, W)], semB)
        wait_idx(srcA, dstA, etA, semA)
        flat_gather(srcA, dstA, etA, fsA, fdA, tA, hA, eA, base_a, semA)
        pltpu.make_async_copy(attB, att_hbm.at[pl.ds(base_b, W)], semB).wait()

        idx_issue(srcB, dstB, etB, base_p, semB)
        wait_gathers(tA, hA, eA, semA)
        compute(dstA, tA, hA, eA, attA)
        pltpu.async_copy(attA, att_hbm.at[pl.ds(base_a, W)], semA)
        wait_idx(srcB, dstB, etB, semB)
        flat_gather(srcB, dstB, etB, fsB, fdB, tB, hB, eB, base_p, semB)
        pltpu.make_async_copy(attA, att_hbm.at[pl.ds(base_a, W)], semA).wait()

    wait_gathers(tB, hB, eB, semB)

    pltpu.sync_copy(tab_v, mparts_hbm.at[pl.ds(wid * NPAD, NPAD)])


@functools.partial(
    pl.kernel,
    out_type=jax.ShapeDtypeStruct((NPAD,), jnp.float32),
    mesh=_mesh,
    compiler_params=_sc_params,
    scratch_types=[
        pltpu.VMEM((NW * SEG,), jnp.float32),
        pltpu.VMEM((SEG,), jnp.float32),
        pltpu.SemaphoreType.DMA,
    ],
)
def _amax_combine_kernel(parts_hbm, out_hbm, buf_v, res_v, sem):
    wid = _wid()
    col = wid * SEG
    for k in range(NW):
        pltpu.sync_copy(parts_hbm.at[pl.ds(k * NPAD + col, SEG)],
                        buf_v.at[pl.ds(k * SEG, SEG)])

    @pl.loop(0, SEG // L)
    def _blk(t):
        m = buf_v[pl.ds(t * L, L)]
        for k in range(1, NW):
            m = jnp.maximum(m, buf_v[pl.ds(k * SEG + t * L, L)])
        res_v[pl.ds(t * L, L)] = jnp.where(m > -1.0e38, m, 0.0)

    pltpu.sync_copy(res_v, out_hbm.at[pl.ds(col, SEG)])


@functools.partial(
    pl.kernel,
    out_type=jax.ShapeDtypeStruct((NW * NPAD,), jnp.float32),
    mesh=_mesh,
    compiler_params=_sc_params,
    scratch_types=[
        pltpu.VMEM((NPAD,), jnp.float32),
        pltpu.VMEM((NPAD,), jnp.float32),
        pltpu.VMEM((W,), jnp.float32),
        pltpu.VMEM((W,), jnp.int32),
        pltpu.VMEM((L,), jnp.int32),
        pltpu.VMEM((L,), jnp.float32),
        pltpu.SemaphoreType.DMA,
    ],
)
def _denom_parts_kernel(att_hbm, dst_hbm, amax_hbm, dparts_hbm,
                        amax_v, dtab_v, att_v, dst_v, kbuf, sbuf, sem):
    wid = _wid()
    base0 = wid * CH
    iota = lax.iota(jnp.int32, L)
    pltpu.sync_copy(amax_hbm, amax_v)

    @pl.loop(0, NPAD // L)
    def _init(i):
        dtab_v[pl.ds(i * L, L)] = jnp.zeros((L,), jnp.float32)

    @pl.loop(0, NWIN)
    def _win(w):
        base = base0 + w * W
        pltpu.sync_copy(att_hbm.at[pl.ds(base, W)], att_v)
        pltpu.sync_copy(dst_hbm.at[pl.ds(base, W)], dst_v)

        @pl.loop(0, W // L)
        def _blk(k):
            sl = pl.ds(k * L, L)
            d = dst_v[sl]
            am = plsc.load_gather(amax_v, [d])
            ex = jnp.exp(att_v[sl] - am)
            sk, sv = plsc.sort_key_val(d, ex)
            for sh in (1, 2, 4, 8):
                kbuf[...] = sk
                sbuf[...] = sv
                pidx = jnp.maximum(iota - sh, 0)
                pk = plsc.load_gather(kbuf, [pidx])
                pv = plsc.load_gather(sbuf, [pidx])
                take = jnp.logical_and(iota >= sh, pk == sk)
                sv = sv + jnp.where(take, pv, 0.0)
            kbuf[...] = sk
            nk = plsc.load_gather(kbuf, [jnp.minimum(iota + 1, L - 1)])
            islast = jnp.logical_or(nk != sk, iota == L - 1)
            plsc.addupdate_scatter(dtab_v, [sk], sv, mask=islast)

    pltpu.sync_copy(dtab_v, dparts_hbm.at[pl.ds(wid * NPAD, NPAD)])


@functools.partial(
    pl.kernel,
    out_type=jax.ShapeDtypeStruct((NPAD,), jnp.float32),
    mesh=_mesh,
    compiler_params=_sc_params,
    scratch_types=[
        pltpu.VMEM((NW * SEG,), jnp.float32),
        pltpu.VMEM((SEG,), jnp.float32),
        pltpu.SemaphoreType.DMA,
    ],
)
def _denom_combine_kernel(parts_hbm, out_hbm, buf_v, res_v, sem):
    wid = _wid()
    col = wid * SEG
    for k in range(NW):
        pltpu.sync_copy(parts_hbm.at[pl.ds(k * NPAD + col, SEG)],
                        buf_v.at[pl.ds(k * SEG, SEG)])

    @pl.loop(0, SEG // L)
    def _blk(t):
        m = buf_v[pl.ds(t * L, L)]
        for k in range(1, NW):
            m = m + buf_v[pl.ds(k * SEG + t * L, L)]
        res_v[pl.ds(t * L, L)] = m

    pltpu.sync_copy(res_v, out_hbm.at[pl.ds(col, SEG)])


@functools.partial(
    pl.kernel,
    out_type=jax.ShapeDtypeStruct((NC * NPAD, D), jnp.float32),
    mesh=_mesh,
    compiler_params=_sc_params,
    scratch_types=[
        pltpu.VMEM((NPAD,), jnp.float32),
        pltpu.VMEM((W,), jnp.int32),
        pltpu.VMEM((W,), jnp.int32),
        pltpu.VMEM((W,), jnp.float32),
        pltpu.VMEM((W,), jnp.float32),
        pltpu.VMEM((W, D), jnp.float32),
        [pltpu.VMEM((L,), jnp.int32) for _ in range(W // L)],
        pltpu.VMEM_SHARED((NPAD, D), jnp.float32),
        pltpu.SemaphoreType.DMA,
    ],
)
def _aggregate_kernel(att_hbm, dst_hbm, src_hbm, amax_hbm, h_hbm,
                      acc_out,
                      amax_v, src_v, dst_v, att_v, exl_v, rows_v,
                      idx2, acc_sh, sem):
    cid = lax.axis_index("c")
    sid = lax.axis_index("s")
    wid = sid * NC + cid
    base0 = wid * CH

    pltpu.sync_copy(amax_hbm, amax_v)

    @pl.loop(0, W)
    def _z(i):
        for q in range(D // L):
            rows_v[i, pl.ds(q * L, L)] = jnp.zeros((L,), jnp.float32)

    stripe = NPAD // NS
    rb = sid * stripe

    @pl.loop(0, stripe // W)
    def _zs(t):
        pltpu.sync_copy(rows_v, acc_sh.at[pl.ds(rb + t * W, W)])

    plsc.subcore_barrier()

    @pl.loop(0, NWIN)
    def _win(w):
        base = base0 + w * W
        pltpu.sync_copy(src_hbm.at[pl.ds(base, W)], src_v)
        pltpu.sync_copy(dst_hbm.at[pl.ds(base, W)], dst_v)
        cp1 = pltpu.async_copy(att_hbm.at[pl.ds(base, W)], att_v, sem)
        cp2 = pltpu.async_copy(h_hbm.at[src_v], rows_v, sem)
        cp1.wait()
        cp2.wait()

        for k in range(W // L):
            sl = pl.ds(k * L, L)
            d = dst_v[sl]
            idx2[k][...] = d
            am = plsc.load_gather(amax_v, [d])
            exl_v[sl] = jnp.exp(att_v[sl] - am)
            for j in range(L):
                e = k * L + j
                spl = plsc.load_gather(
                    exl_v, [jnp.full((L,), e, jnp.int32)])
                for q in range(D // L):
                    qs = pl.ds(q * L, L)
                    rows_v[e, qs] = rows_v[e, qs] * spl
            pltpu.sync_copy(rows_v.at[sl], acc_sh.at[idx2[k]], add=True)

    plsc.subcore_barrier()

    ob = cid * NPAD + rb
    pltpu.sync_copy(acc_sh.at[pl.ds(rb, stripe)],
                    acc_out.at[pl.ds(ob, stripe)])


def _final_body(h_ref, acc_ref, den_ref, w1_ref, w2_ref, o_ref):
    acc = acc_ref[0] + acc_ref[1]
    denom = jnp.maximum(den_ref[...], 1e-16)
    hn = acc / denom
    h = h_ref[...]
    y1 = lax.dot_general(h + hn, w1_ref[...], (((1,), (1,)), ((), ())),
                         preferred_element_type=jnp.float32)
    y2 = lax.dot_general(h * hn, w2_ref[...], (((1,), (1,)), ((), ())),
                         preferred_element_type=jnp.float32)
    o_ref[...] = (jnp.where(y1 > 0, y1, 0.01 * y1)
                  + jnp.where(y2 > 0, y2, 0.01 * y2))


def _final(h, acc_parts, denom, w1, w2):
    BR = 1000
    G = N // BR
    return pl.pallas_call(
        _final_body,
        grid=(G,),
        in_specs=[
            pl.BlockSpec((BR, D), lambda i: (i, 0)),
            pl.BlockSpec((NC, BR, D), lambda i: (0, i, 0)),
            pl.BlockSpec((BR, 1), lambda i: (i, 0)),
            pl.BlockSpec((D, D), lambda i: (0, 0)),
            pl.BlockSpec((D, D), lambda i: (0, 0)),
        ],
        out_specs=pl.BlockSpec((BR, D), lambda i: (i, 0)),
        out_shape=jax.ShapeDtypeStruct((N, D), jnp.float32),
    )(h, acc_parts, denom, w1, w2)


def kernel(nfeat, efeat, edge_index, edge_type, relation_weight,
           W_res, W_res_2):
    src = edge_index[0]
    dst = edge_index[1]
    all_proj = _all_proj(nfeat, relation_weight)
    att, parts = _att_kernel(all_proj, efeat, src, dst, edge_type)
    amax = _amax_combine_kernel(parts)
    dparts = _denom_parts_kernel(att, dst, amax)
    denom = _denom_combine_kernel(dparts)
    acc_flat = _aggregate_kernel(att, dst, src, amax, nfeat)
    acc_parts = acc_flat.reshape(NC, NPAD, D)[:, :N]
    return _final(nfeat, acc_parts, denom.reshape(NPAD, 1)[:N],
                  W_res, W_res_2)
